# initial kernel scaffold (unmeasured)
import jax
import jax.numpy as jnp
from jax import lax
from jax.experimental import pallas as pl
from jax.experimental.pallas import tpu as pltpu

N_DEV = 4
T = 512
D = 1024
V_LOC = 8192
V_TILE = 2048
N_TILES = V_LOC // V_TILE


def kernel(x, W, labels):
    labels2d = labels.reshape(T, 1)

    def body(x_ref, w_ref, lab_ref, out_ref, comm_ref, send_sems, recv_sems):
        my_i = lax.axis_index("i")

        xb = x_ref[...].astype(jnp.bfloat16)
        tgt = lab_ref[...] - my_i * V_LOC

        m_run = None
        for t in range(N_TILES):
            wb = w_ref[:, pl.ds(t * V_TILE, V_TILE)].astype(jnp.bfloat16)
            logits = jnp.dot(xb, wb, preferred_element_type=jnp.float32)
            m_t = jnp.max(logits, axis=1)
            s_t = jnp.sum(jnp.exp(logits - m_t[:, None]), axis=1)
            col = lax.broadcasted_iota(jnp.int32, (T, V_TILE), 1) + t * V_TILE
            c_t = jnp.sum(jnp.where(col == tgt, logits, 0.0), axis=1)
            if m_run is None:
                m_run, s_run, c_run = m_t, s_t, c_t
            else:
                m_new = jnp.maximum(m_run, m_t)
                s_run = s_run * jnp.exp(m_run - m_new) + s_t * jnp.exp(m_t - m_new)
                m_run = m_new
                c_run = c_run + c_t

        comm_ref[my_i, 0, :] = m_run
        comm_ref[my_i, 1, :] = s_run
        comm_ref[my_i, 2, :] = c_run

        barrier_sem = pltpu.get_barrier_semaphore()
        for d in range(1, N_DEV):
            pl.semaphore_signal(
                barrier_sem, inc=1,
                device_id=((my_i + d) % N_DEV,),
                device_id_type=pl.DeviceIdType.MESH,
            )
        pl.semaphore_wait(barrier_sem, N_DEV - 1)

        sends = []
        for d in range(1, N_DEV):
            rdma = pltpu.make_async_remote_copy(
                src_ref=comm_ref.at[my_i],
                dst_ref=comm_ref.at[my_i],
                send_sem=send_sems.at[d - 1],
                recv_sem=recv_sems.at[my_i],
                device_id=((my_i + d) % N_DEV,),
                device_id_type=pl.DeviceIdType.MESH,
            )
            rdma.start()
            sends.append(rdma)

        for d in range(1, N_DEV):
            src = (my_i - d) % N_DEV
            recv = pltpu.make_async_remote_copy(
                src_ref=comm_ref.at[src],
                dst_ref=comm_ref.at[src],
                send_sem=send_sems.at[d - 1],
                recv_sem=recv_sems.at[src],
                device_id=(src,),
                device_id_type=pl.DeviceIdType.MESH,
            )
            recv.wait_recv()
        for rdma in sends:
            rdma.wait_send()

        stats = comm_ref[...]
        M = stats[:, 0, :]
        S = stats[:, 1, :]
        C = stats[:, 2, :]
        m_g = jnp.max(M, axis=0)
        s_g = jnp.sum(S * jnp.exp(M - m_g[None, :]), axis=0)
        c_g = jnp.sum(C, axis=0)
        out_ref[...] = m_g + jnp.log(s_g) - c_g

    return pl.pallas_call(
        body,
        out_shape=jax.ShapeDtypeStruct((T,), jnp.float32),
        in_specs=[
            pl.BlockSpec(memory_space=pltpu.VMEM),
            pl.BlockSpec(memory_space=pltpu.VMEM),
            pl.BlockSpec(memory_space=pltpu.VMEM),
        ],
        out_specs=pl.BlockSpec(memory_space=pltpu.VMEM),
        scratch_shapes=[
            pltpu.VMEM((N_DEV, 8, T), jnp.float32),
            pltpu.SemaphoreType.DMA((N_DEV - 1,)),
            pltpu.SemaphoreType.DMA((N_DEV,)),
        ],
        compiler_params=pltpu.CompilerParams(collective_id=0),
    )(x, W, labels2d)


# baseline (device time: 32599 ns/iter reference)
import jax
import jax.numpy as jnp
from jax import lax
from jax.experimental import pallas as pl
from jax.experimental.pallas import tpu as pltpu

N_DEV = 4
T = 512
D = 1024
V_LOC = 8192
V_TILE = 2048
N_TILES = V_LOC // V_TILE


def kernel(x, W, labels):
    labels2d = labels.reshape(T, 1)

    def body(x_ref, w_ref, lab_ref, out_ref, comm_ref, send_sems, recv_sems):
        my_i = lax.axis_index("i")
        t = pl.program_id(0)
        barrier_sem = pltpu.get_barrier_semaphore()

        xb = x_ref[...].astype(jnp.bfloat16)
        wb = w_ref[...].astype(jnp.bfloat16)
        logits = jnp.dot(xb, wb, preferred_element_type=jnp.float32)
        m_t = jnp.max(logits, axis=1)
        s_t = jnp.sum(jnp.exp(logits - m_t[:, None]), axis=1)
        tgt = lab_ref[...] - my_i * V_LOC
        col = lax.broadcasted_iota(jnp.int32, (T, V_TILE), 1) + t * V_TILE
        c_t = jnp.sum(jnp.where(col == tgt, logits, 0.0), axis=1)

        is_first = t == 0
        m_old = comm_ref[my_i, 0, :]
        s_old = comm_ref[my_i, 1, :]
        c_old = comm_ref[my_i, 2, :]
        m_new = jnp.where(is_first, m_t, jnp.maximum(m_old, m_t))
        s_new = jnp.where(
            is_first, s_t,
            s_old * jnp.exp(m_old - m_new) + s_t * jnp.exp(m_t - m_new),
        )
        c_new = jnp.where(is_first, c_t, c_old + c_t)
        comm_ref[my_i, 0, :] = m_new
        comm_ref[my_i, 1, :] = s_new
        comm_ref[my_i, 2, :] = c_new

        @pl.when(t == N_TILES - 1)
        def _():
            for d in range(1, N_DEV):
                pl.semaphore_signal(
                    barrier_sem, inc=1,
                    device_id=((my_i + d) % N_DEV,),
                    device_id_type=pl.DeviceIdType.MESH,
                )
            pl.semaphore_wait(barrier_sem, N_DEV - 1)

            sends = []
            for d in range(1, N_DEV):
                rdma = pltpu.make_async_remote_copy(
                    src_ref=comm_ref.at[my_i],
                    dst_ref=comm_ref.at[my_i],
                    send_sem=send_sems.at[d - 1],
                    recv_sem=recv_sems.at[my_i],
                    device_id=((my_i + d) % N_DEV,),
                    device_id_type=pl.DeviceIdType.MESH,
                )
                rdma.start()
                sends.append(rdma)

            for d in range(1, N_DEV):
                src = (my_i - d) % N_DEV
                recv = pltpu.make_async_remote_copy(
                    src_ref=comm_ref.at[src],
                    dst_ref=comm_ref.at[src],
                    send_sem=send_sems.at[d - 1],
                    recv_sem=recv_sems.at[src],
                    device_id=(src,),
                    device_id_type=pl.DeviceIdType.MESH,
                )
                recv.wait_recv()
            for rdma in sends:
                rdma.wait_send()

            stats = comm_ref[...]
            M = stats[:, 0, :]
            S = stats[:, 1, :]
            C = stats[:, 2, :]
            m_g = jnp.max(M, axis=0)
            s_g = jnp.sum(S * jnp.exp(M - m_g[None, :]), axis=0)
            c_g = jnp.sum(C, axis=0)
            out_ref[...] = m_g + jnp.log(s_g) - c_g

    return pl.pallas_call(
        body,
        grid=(N_TILES,),
        out_shape=jax.ShapeDtypeStruct((T,), jnp.float32),
        in_specs=[
            pl.BlockSpec((T, D), lambda t: (0, 0)),
            pl.BlockSpec((D, V_TILE), lambda t: (0, t)),
            pl.BlockSpec((T, 1), lambda t: (0, 0)),
        ],
        out_specs=pl.BlockSpec((T,), lambda t: (0,)),
        scratch_shapes=[
            pltpu.VMEM((N_DEV, 8, T), jnp.float32),
            pltpu.SemaphoreType.DMA((N_DEV - 1,)),
            pltpu.SemaphoreType.DMA((N_DEV,)),
        ],
        compiler_params=pltpu.CompilerParams(collective_id=0),
    )(x, W, labels2d)


# device time: 29697 ns/iter; 1.0977x vs baseline; 1.0977x over previous
import jax
import jax.numpy as jnp
from jax import lax
from jax.experimental import pallas as pl
from jax.experimental.pallas import tpu as pltpu

N_DEV = 4
T = 512
D = 1024
V_LOC = 8192
V_TILE = 2048
N_TILES = V_LOC // V_TILE
K_HALF = D // 2


def kernel(x, W, labels):
    labels2d = labels.reshape(T, 1)

    def body(x_ref, wa_ref, wb_ref, lab_ref, out_ref,
             comm_ref, send_sems, recv_sems):
        my_i = lax.axis_index("i")
        t = pl.program_id(0)
        barrier_sem = pltpu.get_barrier_semaphore()

        @pl.when(t == 0)
        def _():
            for d in range(1, N_DEV):
                pl.semaphore_signal(
                    barrier_sem, inc=1,
                    device_id=((my_i + d) % N_DEV,),
                    device_id_type=pl.DeviceIdType.MESH,
                )
            pl.semaphore_wait(barrier_sem, N_DEV - 1)

        xb = x_ref[...].astype(jnp.bfloat16)
        wa = wa_ref[...].astype(jnp.bfloat16)
        wb = wb_ref[...].astype(jnp.bfloat16)
        logits = jnp.dot(xb[:, :K_HALF], wa, preferred_element_type=jnp.float32)
        logits = logits + jnp.dot(
            xb[:, K_HALF:], wb, preferred_element_type=jnp.float32
        )

        s_t = jnp.sum(jnp.exp(logits), axis=1)
        tgt = lab_ref[...] - my_i * V_LOC
        col = lax.broadcasted_iota(jnp.int32, (T, V_TILE), 1) + t * V_TILE
        c_t = jnp.sum(jnp.where(col == tgt, logits, 0.0), axis=1)

        is_first = t == 0
        s_old = comm_ref[my_i, 0, :]
        c_old = comm_ref[my_i, 1, :]
        comm_ref[my_i, 0, :] = jnp.where(is_first, s_t, s_old + s_t)
        comm_ref[my_i, 1, :] = jnp.where(is_first, c_t, c_old + c_t)

        @pl.when(t == N_TILES - 1)
        def _():
            sends = []
            for d in range(1, N_DEV):
                rdma = pltpu.make_async_remote_copy(
                    src_ref=comm_ref.at[my_i],
                    dst_ref=comm_ref.at[my_i],
                    send_sem=send_sems.at[d - 1],
                    recv_sem=recv_sems.at[my_i],
                    device_id=((my_i + d) % N_DEV,),
                    device_id_type=pl.DeviceIdType.MESH,
                )
                rdma.start()
                sends.append(rdma)

            for d in range(1, N_DEV):
                src = (my_i - d) % N_DEV
                recv = pltpu.make_async_remote_copy(
                    src_ref=comm_ref.at[src],
                    dst_ref=comm_ref.at[src],
                    send_sem=send_sems.at[d - 1],
                    recv_sem=recv_sems.at[src],
                    device_id=(src,),
                    device_id_type=pl.DeviceIdType.MESH,
                )
                recv.wait_recv()
            for rdma in sends:
                rdma.wait_send()

            stats = comm_ref[...]
            s_g = jnp.sum(stats[:, 0, :], axis=0)
            c_g = jnp.sum(stats[:, 1, :], axis=0)
            out_ref[...] = jnp.log(s_g) - c_g

    return pl.pallas_call(
        body,
        grid=(N_TILES,),
        out_shape=jax.ShapeDtypeStruct((T,), jnp.float32),
        in_specs=[
            pl.BlockSpec((T, D), lambda t: (0, 0)),
            pl.BlockSpec((K_HALF, V_TILE), lambda t: (0, t)),
            pl.BlockSpec((K_HALF, V_TILE), lambda t: (1, t)),
            pl.BlockSpec((T, 1), lambda t: (0, 0)),
        ],
        out_specs=pl.BlockSpec((T,), lambda t: (0,)),
        scratch_shapes=[
            pltpu.VMEM((N_DEV, 8, T), jnp.float32),
            pltpu.SemaphoreType.DMA((N_DEV - 1,)),
            pltpu.SemaphoreType.DMA((N_DEV,)),
        ],
        compiler_params=pltpu.CompilerParams(collective_id=0),
    )(x, W, W, labels2d)


# device time: 29481 ns/iter; 1.1058x vs baseline; 1.0073x over previous
import jax
import jax.numpy as jnp
from jax import lax
from jax.experimental import pallas as pl
from jax.experimental.pallas import tpu as pltpu

N_DEV = 4
T = 512
D = 1024
V_LOC = 8192
V_TILE = 2048
N_TILES = V_LOC // V_TILE
K_HALF = D // 2


def kernel(x, W, labels):
    labels2d = labels.reshape(T, 1)

    def body(x_ref, wa_ref, wb_ref, lab_ref, out_ref,
             comm_ref, logits_buf, send_sems, recv_sems):
        my_i = lax.axis_index("i")
        t = pl.program_id(0)
        barrier_sem = pltpu.get_barrier_semaphore()

        @pl.when(t == 0)
        def _():
            for d in range(1, N_DEV):
                pl.semaphore_signal(
                    barrier_sem, inc=1,
                    device_id=((my_i + d) % N_DEV,),
                    device_id_type=pl.DeviceIdType.MESH,
                )
            pl.semaphore_wait(barrier_sem, N_DEV - 1)

        tgt = lab_ref[...] - my_i * V_LOC

        def stats(logits, tile_idx):
            s_t = jnp.sum(jnp.exp(logits), axis=1)
            col = (lax.broadcasted_iota(jnp.int32, (T, V_TILE), 1)
                   + tile_idx * V_TILE)
            c_t = jnp.sum(jnp.where(col == tgt, logits, 0.0), axis=1)
            return s_t, c_t

        prev = logits_buf[...]
        s_t, c_t = stats(prev, t - 1)
        s_old = comm_ref[my_i, 0, :]
        c_old = comm_ref[my_i, 1, :]
        zero = jnp.zeros((T,), jnp.float32)
        comm_ref[my_i, 0, :] = jnp.where(
            t == 0, zero, jnp.where(t == 1, s_t, s_old + s_t))
        comm_ref[my_i, 1, :] = jnp.where(
            t == 0, zero, jnp.where(t == 1, c_t, c_old + c_t))

        xb = x_ref[...].astype(jnp.bfloat16)
        wa = wa_ref[...].astype(jnp.bfloat16)
        wb = wb_ref[...].astype(jnp.bfloat16)
        logits = jnp.dot(xb[:, :K_HALF], wa, preferred_element_type=jnp.float32)
        logits = logits + jnp.dot(
            xb[:, K_HALF:], wb, preferred_element_type=jnp.float32
        )
        logits_buf[...] = logits

        @pl.when(t == N_TILES - 1)
        def _():
            s_l, c_l = stats(logits, N_TILES - 1)
            comm_ref[my_i, 0, :] = comm_ref[my_i, 0, :] + s_l
            comm_ref[my_i, 1, :] = comm_ref[my_i, 1, :] + c_l

            sends = []
            for d in range(1, N_DEV):
                rdma = pltpu.make_async_remote_copy(
                    src_ref=comm_ref.at[my_i],
                    dst_ref=comm_ref.at[my_i],
                    send_sem=send_sems.at[d - 1],
                    recv_sem=recv_sems.at[my_i],
                    device_id=((my_i + d) % N_DEV,),
                    device_id_type=pl.DeviceIdType.MESH,
                )
                rdma.start()
                sends.append(rdma)

            for d in range(1, N_DEV):
                src = (my_i - d) % N_DEV
                recv = pltpu.make_async_remote_copy(
                    src_ref=comm_ref.at[src],
                    dst_ref=comm_ref.at[src],
                    send_sem=send_sems.at[d - 1],
                    recv_sem=recv_sems.at[src],
                    device_id=(src,),
                    device_id_type=pl.DeviceIdType.MESH,
                )
                recv.wait_recv()
            for rdma in sends:
                rdma.wait_send()

            stats_all = comm_ref[...]
            s_g = jnp.sum(stats_all[:, 0, :], axis=0)
            c_g = jnp.sum(stats_all[:, 1, :], axis=0)
            out_ref[...] = jnp.log(s_g) - c_g

    return pl.pallas_call(
        body,
        grid=(N_TILES,),
        out_shape=jax.ShapeDtypeStruct((T,), jnp.float32),
        in_specs=[
            pl.BlockSpec((T, D), lambda t: (0, 0)),
            pl.BlockSpec((K_HALF, V_TILE), lambda t: (0, t)),
            pl.BlockSpec((K_HALF, V_TILE), lambda t: (1, t)),
            pl.BlockSpec((T, 1), lambda t: (0, 0)),
        ],
        out_specs=pl.BlockSpec((T,), lambda t: (0,)),
        scratch_shapes=[
            pltpu.VMEM((N_DEV, 8, T), jnp.float32),
            pltpu.VMEM((T, V_TILE), jnp.float32),
            pltpu.SemaphoreType.DMA((N_DEV - 1,)),
            pltpu.SemaphoreType.DMA((N_DEV,)),
        ],
        compiler_params=pltpu.CompilerParams(collective_id=0),
    )(x, W, W, labels2d)
